# baseline (speedup 1.0000x reference)
import jax
import jax.numpy as jnp
from jax import lax
from jax.experimental import pallas as pl
from jax.experimental.pallas import tpu as pltpu


def _round_up(x, m):
    return (x + m - 1) // m * m


def _decoder_kernel(ids_ref, emb_hbm, wih_hbm, whh_hbm, b_ref, wout_hbm,
                    bout_ref, h0_hbm, c0_hbm,
                    scores_hbm, hN_ref, cN_ref,
                    emb_v, wih_v, whh_v, wout_v, h0_v, c0_v, h_all, sc_v,
                    sems):
    LB = ids_ref.shape[0]
    B, H = h0_v.shape
    L = LB // B
    V, E = emb_v.shape
    EC = E // 2
    KC = H // 4
    NM = 8
    MC = LB // NM

    def cp(src, dst, i):
        return pltpu.make_async_copy(src, dst, sems.at[i])

    def wih_cp(j):
        return cp(wih_hbm.at[pl.ds(j * EC, EC)], wih_v.at[pl.ds(j * EC, EC)],
                  1 + j)

    def whh_cp(k):
        return cp(whh_hbm.at[pl.ds(k * KC, KC)], whh_v.at[pl.ds(k * KC, KC)],
                  3 + k)

    cp(emb_hbm, emb_v, 0).start()
    wih_cp(0).start()
    wih_cp(1).start()
    for k in range(4):
        whh_cp(k).start()
    cp(wout_hbm, wout_v, 7).start()
    cp(h0_hbm, h0_v, 8).start()
    cp(c0_hbm, c0_v, 9).start()

    cp(emb_hbm, emb_v, 0).wait()
    iota = lax.broadcasted_iota(jnp.int32, (LB, V), 1)
    onehot = (ids_ref[...] == iota).astype(jnp.float32)
    x = jnp.dot(onehot, emb_v[...], preferred_element_type=jnp.float32)

    wih_cp(0).wait()
    gates = (jnp.dot(x[:, 0:EC], wih_v[0:EC, :],
                     preferred_element_type=jnp.float32) + b_ref[...])
    wih_cp(1).wait()
    gates = gates + jnp.dot(x[:, EC:], wih_v[EC:, :],
                            preferred_element_type=jnp.float32)

    cp(h0_hbm, h0_v, 8).wait()
    cp(c0_hbm, c0_v, 9).wait()
    h = h0_v[...]
    c = c0_v[...]

    def lstm_cell(g, c):
        i_g = jax.nn.sigmoid(g[:, 0 * H:1 * H])
        f_g = jax.nn.sigmoid(g[:, 1 * H:2 * H])
        g_g = jnp.tanh(g[:, 2 * H:3 * H])
        o_g = jax.nn.sigmoid(g[:, 3 * H:4 * H])
        c = f_g * c + i_g * g_g
        return o_g * jnp.tanh(c), c

    g = gates[0:B, :]
    whh_cp(0).wait()
    g = g + jnp.dot(h[:, 0:KC], whh_v[0:KC, :],
                    preferred_element_type=jnp.float32)
    whh_cp(1).wait()
    g = g + jnp.dot(h[:, KC:2 * KC], whh_v[KC:2 * KC, :],
                    preferred_element_type=jnp.float32)
    whh_cp(2).wait()
    g = g + jnp.dot(h[:, 2 * KC:3 * KC], whh_v[2 * KC:3 * KC, :],
                    preferred_element_type=jnp.float32)
    whh_cp(3).wait()
    g = g + jnp.dot(h[:, 3 * KC:], whh_v[3 * KC:, :],
                    preferred_element_type=jnp.float32)
    h, c = lstm_cell(g, c)
    h_all[0:B, :] = h

    for t in range(1, L):
        g = gates[t * B:(t + 1) * B, :] + jnp.dot(
            h, whh_v[...], preferred_element_type=jnp.float32)
        h, c = lstm_cell(g, c)
        h_all[t * B:(t + 1) * B, :] = h

    hN_ref[...] = h
    cN_ref[...] = c

    cp(wout_hbm, wout_v, 7).wait()
    for m in range(NM):
        sl = pl.ds(m * MC, MC)
        sc_v[sl, :] = (jnp.dot(h_all[m * MC:(m + 1) * MC, :], wout_v[...],
                               preferred_element_type=jnp.float32)
                       + bout_ref[...])
        cp(sc_v.at[sl], scores_hbm.at[sl], 10 + m).start()
    for m in range(NM):
        cp(sc_v.at[pl.ds(m * MC, MC)], scores_hbm.at[pl.ds(m * MC, MC)],
           10 + m).wait()


def kernel(input_ids, emb, w_ih_T, w_hh_T, b_lstm, w_out_T, b_out, h0, c0):
    L, B = input_ids.shape
    V, E = emb.shape
    H = w_hh_T.shape[0]
    Vo = w_out_T.shape[1]

    Bp = _round_up(B, 8)
    Vp = _round_up(Vo, 128)

    if Bp != B:
        ids_p = jnp.pad(input_ids, ((0, 0), (0, Bp - B)), constant_values=-1)
        h0p = jnp.pad(h0[0], ((0, Bp - B), (0, 0)))
        c0p = jnp.pad(c0[0], ((0, Bp - B), (0, 0)))
    else:
        ids_p, h0p, c0p = input_ids, h0[0], c0[0]
    if Vp != Vo:
        w_out_p = jnp.pad(w_out_T, ((0, 0), (0, Vp - Vo)))
        b_out_p = jnp.pad(b_out, ((0, 0), (0, Vp - Vo)))
    else:
        w_out_p, b_out_p = w_out_T, b_out
    ids_flat = ids_p.reshape(L * Bp, 1)

    anyspec = pl.BlockSpec(memory_space=pl.ANY)
    vmem = pl.BlockSpec(memory_space=pltpu.VMEM)

    scores_flat, h_n_p, c_n_p = pl.pallas_call(
        _decoder_kernel,
        out_shape=(
            jax.ShapeDtypeStruct((L * Bp, Vp), jnp.float32),
            jax.ShapeDtypeStruct((Bp, H), jnp.float32),
            jax.ShapeDtypeStruct((Bp, H), jnp.float32),
        ),
        in_specs=[vmem, anyspec, anyspec, anyspec, vmem, anyspec, vmem,
                  anyspec, anyspec],
        out_specs=(anyspec, vmem, vmem),
        scratch_shapes=[
            pltpu.VMEM((V, E), jnp.float32),
            pltpu.VMEM((E, 4 * H), jnp.float32),
            pltpu.VMEM((H, 4 * H), jnp.float32),
            pltpu.VMEM((H, Vp), jnp.float32),
            pltpu.VMEM((Bp, H), jnp.float32),
            pltpu.VMEM((Bp, H), jnp.float32),
            pltpu.VMEM((L * Bp, H), jnp.float32),
            pltpu.VMEM((L * Bp, Vp), jnp.float32),
            pltpu.SemaphoreType.DMA((18,)),
        ],
    )(ids_flat, emb, w_ih_T, w_hh_T, b_lstm, w_out_p, b_out_p, h0p, c0p)

    scores = scores_flat.reshape(L, Bp, Vp)[:, :B, :Vo]
    h_n = h_n_p[:B][None]
    c_n = c_n_p[:B][None]
    return scores, (h_n, c_n)


# speedup vs baseline: 1.0386x; 1.0386x over previous
import jax
import jax.numpy as jnp
from jax import lax
from jax.experimental import pallas as pl
from jax.experimental.pallas import tpu as pltpu


def _round_up(x, m):
    return (x + m - 1) // m * m


def _decoder_kernel(ids_ref, emb_hbm, wih_hbm, whh_hbm, b_ref, wout_hbm,
                    bout_ref, h0_ref, c0_ref,
                    scores_hbm, hN_ref, cN_ref,
                    emb_v, wih_v, whh_v, wout_v, h_all, sc_v, sems):
    LB = ids_ref.shape[0]
    B, H = h0_ref.shape
    L = LB // B
    V, E = emb_v.shape
    EC = E // 2
    KC = H // 2
    NM = 4
    MC = LB // NM

    def cp(src, dst, i):
        return pltpu.make_async_copy(src, dst, sems.at[i])

    def wih_cp(j):
        return cp(wih_hbm.at[pl.ds(j * EC, EC)], wih_v.at[pl.ds(j * EC, EC)],
                  1 + j)

    def whh_cp(k):
        return cp(whh_hbm.at[pl.ds(k * KC, KC)], whh_v.at[pl.ds(k * KC, KC)],
                  3 + k)

    cp(emb_hbm, emb_v, 0).start()
    wih_cp(0).start()
    wih_cp(1).start()
    for k in range(2):
        whh_cp(k).start()
    cp(wout_hbm, wout_v, 7).start()

    cp(emb_hbm, emb_v, 0).wait()
    iota = lax.broadcasted_iota(jnp.int32, (LB, V), 1)
    onehot = (ids_ref[...] == iota).astype(jnp.float32)
    x = jnp.dot(onehot, emb_v[...], preferred_element_type=jnp.float32)

    wih_cp(0).wait()
    gates = (jnp.dot(x[:, 0:EC], wih_v[0:EC, :],
                     preferred_element_type=jnp.float32) + b_ref[...])
    wih_cp(1).wait()
    gates = gates + jnp.dot(x[:, EC:], wih_v[EC:, :],
                            preferred_element_type=jnp.float32)

    h = h0_ref[...]
    c = c0_ref[...]

    def lstm_cell(g, c):
        i_g = jax.nn.sigmoid(g[:, 0 * H:1 * H])
        f_g = jax.nn.sigmoid(g[:, 1 * H:2 * H])
        g_g = jnp.tanh(g[:, 2 * H:3 * H])
        o_g = jax.nn.sigmoid(g[:, 3 * H:4 * H])
        c = f_g * c + i_g * g_g
        return o_g * jnp.tanh(c), c

    g = gates[0:B, :]
    whh_cp(0).wait()
    g = g + jnp.dot(h[:, 0:KC], whh_v[0:KC, :],
                    preferred_element_type=jnp.float32)
    whh_cp(1).wait()
    g = g + jnp.dot(h[:, KC:], whh_v[KC:, :],
                    preferred_element_type=jnp.float32)
    h, c = lstm_cell(g, c)
    h_all[0:B, :] = h

    for t in range(1, L):
        g = gates[t * B:(t + 1) * B, :] + jnp.dot(
            h, whh_v[...], preferred_element_type=jnp.float32)
        h, c = lstm_cell(g, c)
        h_all[t * B:(t + 1) * B, :] = h

    hN_ref[...] = h
    cN_ref[...] = c

    cp(wout_hbm, wout_v, 7).wait()
    for m in range(NM):
        sl = pl.ds(m * MC, MC)
        sc_v[sl, :] = (jnp.dot(h_all[m * MC:(m + 1) * MC, :], wout_v[...],
                               preferred_element_type=jnp.float32)
                       + bout_ref[...])
        cp(sc_v.at[sl], scores_hbm.at[sl], 8 + m).start()
    for m in range(NM):
        cp(sc_v.at[pl.ds(m * MC, MC)], scores_hbm.at[pl.ds(m * MC, MC)],
           8 + m).wait()


def kernel(input_ids, emb, w_ih_T, w_hh_T, b_lstm, w_out_T, b_out, h0, c0):
    L, B = input_ids.shape
    V, E = emb.shape
    H = w_hh_T.shape[0]
    Vo = w_out_T.shape[1]

    Bp = _round_up(B, 8)
    Vp = _round_up(Vo, 128)

    if Bp != B:
        ids_p = jnp.pad(input_ids, ((0, 0), (0, Bp - B)), constant_values=-1)
        h0p = jnp.pad(h0[0], ((0, Bp - B), (0, 0)))
        c0p = jnp.pad(c0[0], ((0, Bp - B), (0, 0)))
    else:
        ids_p, h0p, c0p = input_ids, h0[0], c0[0]
    if Vp != Vo:
        w_out_p = jnp.pad(w_out_T, ((0, 0), (0, Vp - Vo)))
        b_out_p = jnp.pad(b_out, ((0, 0), (0, Vp - Vo)))
    else:
        w_out_p, b_out_p = w_out_T, b_out
    ids_flat = ids_p.reshape(L * Bp, 1)

    anyspec = pl.BlockSpec(memory_space=pl.ANY)
    vmem = pl.BlockSpec(memory_space=pltpu.VMEM)

    scores_flat, h_n_p, c_n_p = pl.pallas_call(
        _decoder_kernel,
        out_shape=(
            jax.ShapeDtypeStruct((L * Bp, Vp), jnp.float32),
            jax.ShapeDtypeStruct((Bp, H), jnp.float32),
            jax.ShapeDtypeStruct((Bp, H), jnp.float32),
        ),
        in_specs=[vmem, anyspec, anyspec, anyspec, vmem, anyspec, vmem,
                  vmem, vmem],
        out_specs=(anyspec, vmem, vmem),
        scratch_shapes=[
            pltpu.VMEM((V, E), jnp.float32),
            pltpu.VMEM((E, 4 * H), jnp.float32),
            pltpu.VMEM((H, 4 * H), jnp.float32),
            pltpu.VMEM((H, Vp), jnp.float32),
            pltpu.VMEM((L * Bp, H), jnp.float32),
            pltpu.VMEM((L * Bp, Vp), jnp.float32),
            pltpu.SemaphoreType.DMA((12,)),
        ],
    )(ids_flat, emb, w_ih_T, w_hh_T, b_lstm, w_out_p, b_out_p, h0p, c0p)

    scores = scores_flat.reshape(L, Bp, Vp)[:, :B, :Vo]
    h_n = h_n_p[:B][None]
    c_n = c_n_p[:B][None]
    return scores, (h_n, c_n)


# speedup vs baseline: 1.0583x; 1.0190x over previous
import jax
import jax.numpy as jnp
from jax import lax
from jax.experimental import pallas as pl
from jax.experimental.pallas import tpu as pltpu


def _round_up(x, m):
    return (x + m - 1) // m * m


def _decoder_kernel(ids_ref, emb_hbm, wih_hbm, whh_hbm, b_ref, wout_hbm,
                    bout_ref, h0_ref, c0_ref,
                    scores_hbm, hN_hbm, cN_hbm,
                    emb_v, wih_v, whh_v, wout_v, h_all, sc_v, hN_v, cN_v,
                    sems):
    LB = ids_ref.shape[0]
    B, H = hN_v.shape
    L = LB // B
    V, E = emb_v.shape
    EC = E // 2
    KC = H // 2
    NM = 4
    MC = LB // NM

    def cp(src, dst, i):
        return pltpu.make_async_copy(src, dst, sems.at[i])

    def wih_cp(j):
        return cp(wih_hbm.at[pl.ds(j * EC, EC)], wih_v.at[pl.ds(j * EC, EC)],
                  1 + j)

    def whh_cp(k):
        return cp(whh_hbm.at[pl.ds(k * KC, KC)], whh_v.at[pl.ds(k * KC, KC)],
                  3 + k)

    cp(emb_hbm, emb_v, 0).start()
    wih_cp(0).start()
    wih_cp(1).start()
    for k in range(2):
        whh_cp(k).start()
    cp(wout_hbm, wout_v, 7).start()

    cp(emb_hbm, emb_v, 0).wait()
    iota = lax.broadcasted_iota(jnp.int32, (LB, V), 1)
    onehot = (ids_ref[...] == iota).astype(jnp.float32)
    x = jnp.dot(onehot, emb_v[...], preferred_element_type=jnp.float32)

    wih_cp(0).wait()
    gates = (jnp.dot(x[:, 0:EC], wih_v[0:EC, :],
                     preferred_element_type=jnp.float32) + b_ref[...])
    wih_cp(1).wait()
    gates = gates + jnp.dot(x[:, EC:], wih_v[EC:, :],
                            preferred_element_type=jnp.float32)

    h = h0_ref[...]
    c = c0_ref[...]

    def lstm_cell(g, c):
        i_g = jax.nn.sigmoid(g[:, 0 * H:1 * H])
        f_g = jax.nn.sigmoid(g[:, 1 * H:2 * H])
        g_g = jnp.tanh(g[:, 2 * H:3 * H])
        o_g = jax.nn.sigmoid(g[:, 3 * H:4 * H])
        c = f_g * c + i_g * g_g
        return o_g * jnp.tanh(c), c

    g = gates[0:B, :]
    whh_cp(0).wait()
    g = g + jnp.dot(h[:, 0:KC], whh_v[0:KC, :],
                    preferred_element_type=jnp.float32)
    whh_cp(1).wait()
    g = g + jnp.dot(h[:, KC:], whh_v[KC:, :],
                    preferred_element_type=jnp.float32)
    h, c = lstm_cell(g, c)
    h_all[0:B, :] = h

    for t in range(1, L):
        g = gates[t * B:(t + 1) * B, :] + jnp.dot(
            h, whh_v[...], preferred_element_type=jnp.float32)
        h, c = lstm_cell(g, c)
        h_all[t * B:(t + 1) * B, :] = h

    hN_v[...] = h
    cN_v[...] = c
    cp(hN_v, hN_hbm, 12).start()
    cp(cN_v, cN_hbm, 13).start()

    cp(wout_hbm, wout_v, 7).wait()
    for m in range(NM):
        sl = pl.ds(m * MC, MC)
        sc_v[sl, :] = (jnp.dot(h_all[m * MC:(m + 1) * MC, :], wout_v[...],
                               preferred_element_type=jnp.float32)
                       + bout_ref[...])
        cp(sc_v.at[sl], scores_hbm.at[sl], 8 + m).start()
    for m in range(NM):
        cp(sc_v.at[pl.ds(m * MC, MC)], scores_hbm.at[pl.ds(m * MC, MC)],
           8 + m).wait()
    cp(hN_v, hN_hbm, 12).wait()
    cp(cN_v, cN_hbm, 13).wait()


def kernel(input_ids, emb, w_ih_T, w_hh_T, b_lstm, w_out_T, b_out, h0, c0):
    L, B = input_ids.shape
    V, E = emb.shape
    H = w_hh_T.shape[0]
    Vo = w_out_T.shape[1]

    Bp = _round_up(B, 8)
    Vp = _round_up(Vo, 128)

    if Bp != B:
        ids_p = jnp.pad(input_ids, ((0, 0), (0, Bp - B)), constant_values=-1)
        h0p = jnp.pad(h0[0], ((0, Bp - B), (0, 0)))
        c0p = jnp.pad(c0[0], ((0, Bp - B), (0, 0)))
    else:
        ids_p, h0p, c0p = input_ids, h0[0], c0[0]
    if Vp != Vo:
        w_out_p = jnp.pad(w_out_T, ((0, 0), (0, Vp - Vo)))
        b_out_p = jnp.pad(b_out, ((0, 0), (0, Vp - Vo)))
    else:
        w_out_p, b_out_p = w_out_T, b_out
    ids_flat = ids_p.reshape(L * Bp, 1)

    anyspec = pl.BlockSpec(memory_space=pl.ANY)
    vmem = pl.BlockSpec(memory_space=pltpu.VMEM)

    scores_flat, h_n_p, c_n_p = pl.pallas_call(
        _decoder_kernel,
        out_shape=(
            jax.ShapeDtypeStruct((L * Bp, Vp), jnp.float32),
            jax.ShapeDtypeStruct((Bp, H), jnp.float32),
            jax.ShapeDtypeStruct((Bp, H), jnp.float32),
        ),
        in_specs=[vmem, anyspec, anyspec, anyspec, vmem, anyspec, vmem,
                  vmem, vmem],
        out_specs=(anyspec, anyspec, anyspec),
        scratch_shapes=[
            pltpu.VMEM((V, E), jnp.float32),
            pltpu.VMEM((E, 4 * H), jnp.float32),
            pltpu.VMEM((H, 4 * H), jnp.float32),
            pltpu.VMEM((H, Vp), jnp.float32),
            pltpu.VMEM((L * Bp, H), jnp.float32),
            pltpu.VMEM((L * Bp, Vp), jnp.float32),
            pltpu.VMEM((Bp, H), jnp.float32),
            pltpu.VMEM((Bp, H), jnp.float32),
            pltpu.SemaphoreType.DMA((14,)),
        ],
    )(ids_flat, emb, w_ih_T, w_hh_T, b_lstm, w_out_p, b_out_p, h0p, c0p)

    scores = scores_flat.reshape(L, Bp, Vp)[:, :B, :Vo]
    h_n = h_n_p[:B][None]
    c_n = c_n_p[:B][None]
    return scores, (h_n, c_n)


# speedup vs baseline: 1.0596x; 1.0012x over previous
import jax
import jax.numpy as jnp
from jax import lax
from jax.experimental import pallas as pl
from jax.experimental.pallas import tpu as pltpu


def _round_up(x, m):
    return (x + m - 1) // m * m


def _decoder_kernel(ids_ref, emb_hbm, wih_hbm, whh_hbm, b_ref, wout_hbm,
                    bout_ref, h0_ref, c0_ref,
                    scores_hbm, hN_hbm, cN_hbm,
                    emb_v, wih_v, whh_v, wout_v, h_all, sc_v, hN_v, cN_v,
                    sems):
    LB = ids_ref.shape[0]
    B, H = hN_v.shape
    L = LB // B
    V, E = emb_v.shape
    EC = E // 2
    KC = H // 2
    NM = 4
    MC = LB // NM

    def cp(src, dst, i):
        return pltpu.make_async_copy(src, dst, sems.at[i])

    def wih_cp(j):
        return cp(wih_hbm.at[pl.ds(j * EC, EC)], wih_v.at[pl.ds(j * EC, EC)],
                  1 + j)

    def whh_cp(k):
        return cp(whh_hbm.at[pl.ds(k * KC, KC)], whh_v.at[pl.ds(k * KC, KC)],
                  3 + k)

    cp(emb_hbm, emb_v, 0).start()
    wih_cp(0).start()
    wih_cp(1).start()
    for k in range(2):
        whh_cp(k).start()

    cp(emb_hbm, emb_v, 0).wait()
    iota = lax.broadcasted_iota(jnp.int32, (LB, V), 1)
    onehot = (ids_ref[...] == iota).astype(jnp.float32)
    x = jnp.dot(onehot, emb_v[...], preferred_element_type=jnp.float32)

    wih_cp(0).wait()
    gates = (jnp.dot(x[:, 0:EC], wih_v[0:EC, :],
                     preferred_element_type=jnp.float32) + b_ref[...])
    wih_cp(1).wait()
    gates = gates + jnp.dot(x[:, EC:], wih_v[EC:, :],
                            preferred_element_type=jnp.float32)

    h = h0_ref[...]
    c = c0_ref[...]

    def lstm_cell(g, c):
        i_g = jax.nn.sigmoid(g[:, 0 * H:1 * H])
        f_g = jax.nn.sigmoid(g[:, 1 * H:2 * H])
        g_g = jnp.tanh(g[:, 2 * H:3 * H])
        o_g = jax.nn.sigmoid(g[:, 3 * H:4 * H])
        c = f_g * c + i_g * g_g
        return o_g * jnp.tanh(c), c

    g = gates[0:B, :]
    whh_cp(0).wait()
    cp(wout_hbm, wout_v, 7).start()
    g = g + jnp.dot(h[:, 0:KC], whh_v[0:KC, :],
                    preferred_element_type=jnp.float32)
    whh_cp(1).wait()
    g = g + jnp.dot(h[:, KC:], whh_v[KC:, :],
                    preferred_element_type=jnp.float32)
    h, c = lstm_cell(g, c)
    h_all[0:B, :] = h

    for t in range(1, L):
        g = gates[t * B:(t + 1) * B, :] + jnp.dot(
            h, whh_v[...], preferred_element_type=jnp.float32)
        h, c = lstm_cell(g, c)
        h_all[t * B:(t + 1) * B, :] = h

    hN_v[...] = h
    cN_v[...] = c
    cp(hN_v, hN_hbm, 12).start()
    cp(cN_v, cN_hbm, 13).start()

    cp(wout_hbm, wout_v, 7).wait()
    for m in range(NM):
        sl = pl.ds(m * MC, MC)
        sc_v[sl, :] = (jnp.dot(h_all[m * MC:(m + 1) * MC, :], wout_v[...],
                               preferred_element_type=jnp.float32)
                       + bout_ref[...])
        cp(sc_v.at[sl], scores_hbm.at[sl], 8 + m).start()
    for m in range(NM):
        cp(sc_v.at[pl.ds(m * MC, MC)], scores_hbm.at[pl.ds(m * MC, MC)],
           8 + m).wait()
    cp(hN_v, hN_hbm, 12).wait()
    cp(cN_v, cN_hbm, 13).wait()


def kernel(input_ids, emb, w_ih_T, w_hh_T, b_lstm, w_out_T, b_out, h0, c0):
    L, B = input_ids.shape
    V, E = emb.shape
    H = w_hh_T.shape[0]
    Vo = w_out_T.shape[1]

    Bp = _round_up(B, 8)
    Vp = _round_up(Vo, 128)

    if Bp != B:
        ids_p = jnp.pad(input_ids, ((0, 0), (0, Bp - B)), constant_values=-1)
        h0p = jnp.pad(h0[0], ((0, Bp - B), (0, 0)))
        c0p = jnp.pad(c0[0], ((0, Bp - B), (0, 0)))
    else:
        ids_p, h0p, c0p = input_ids, h0[0], c0[0]
    if Vp != Vo:
        w_out_p = jnp.pad(w_out_T, ((0, 0), (0, Vp - Vo)))
        b_out_p = jnp.pad(b_out, ((0, 0), (0, Vp - Vo)))
    else:
        w_out_p, b_out_p = w_out_T, b_out
    ids_flat = ids_p.reshape(L * Bp, 1)

    anyspec = pl.BlockSpec(memory_space=pl.ANY)
    vmem = pl.BlockSpec(memory_space=pltpu.VMEM)

    scores_flat, h_n_p, c_n_p = pl.pallas_call(
        _decoder_kernel,
        out_shape=(
            jax.ShapeDtypeStruct((L * Bp, Vp), jnp.float32),
            jax.ShapeDtypeStruct((Bp, H), jnp.float32),
            jax.ShapeDtypeStruct((Bp, H), jnp.float32),
        ),
        in_specs=[vmem, anyspec, anyspec, anyspec, vmem, anyspec, vmem,
                  vmem, vmem],
        out_specs=(anyspec, anyspec, anyspec),
        scratch_shapes=[
            pltpu.VMEM((V, E), jnp.float32),
            pltpu.VMEM((E, 4 * H), jnp.float32),
            pltpu.VMEM((H, 4 * H), jnp.float32),
            pltpu.VMEM((H, Vp), jnp.float32),
            pltpu.VMEM((L * Bp, H), jnp.float32),
            pltpu.VMEM((L * Bp, Vp), jnp.float32),
            pltpu.VMEM((Bp, H), jnp.float32),
            pltpu.VMEM((Bp, H), jnp.float32),
            pltpu.SemaphoreType.DMA((14,)),
        ],
    )(ids_flat, emb, w_ih_T, w_hh_T, b_lstm, w_out_p, b_out_p, h0p, c0p)

    scores = scores_flat.reshape(L, Bp, Vp)[:, :B, :Vo]
    h_n = h_n_p[:B][None]
    c_n = c_n_p[:B][None]
    return scores, (h_n, c_n)
